# baseline (speedup 1.0000x reference)
import functools

import jax
import jax.numpy as jnp
import numpy as np
from jax import lax
from jax.experimental import pallas as pl
from jax.experimental.pallas import tpu as pltpu
from jax.experimental.pallas import tpu_sc as plsc

LAT_MIN, LAT_MAX = -90.0, 90.0
LON_MIN, LON_MAX = -180.0, 180.0
BINS = 100
EMBED_DIM = 64
BATCH = 16384

NC, NS, L = 2, 16, 16
NW = NC * NS
CHUNK = BATCH // NW
GATHER = 128
NGATHER = 2 * CHUNK // GATHER
GROUPS_PER_GATHER = GATHER // (2 * L)
NPAIR = NGATHER // 2

_PAD = 112


def _buckets(lo, hi):
    b = np.linspace(lo, hi, BINS - 1).astype(np.float32)
    return np.pad(b, (0, _PAD - (BINS - 1)), constant_values=b[-1])


BKS = np.stack([_buckets(LAT_MIN, LAT_MAX), _buckets(LON_MIN, LON_MAX)])


def _digitize(x, bk_ref, lo, hi):
    scale = float(BINS - 2) / (hi - lo)
    t = (x - lo) * scale
    t = jnp.minimum(jnp.maximum(t, -1.0), float(BINS + 1))
    i = t.astype(jnp.int32) + 1
    i = jnp.clip(i, 0, BINS - 1)
    for _ in range(2):
        b_lo = plsc.load_gather(bk_ref, [jnp.clip(i - 1, 0, BINS - 2)])
        b_hi = plsc.load_gather(bk_ref, [jnp.clip(i, 0, BINS - 2)])
        dec = (i > 0) & (b_lo > x)
        inc = (i < BINS - 1) & (b_hi <= x)
        i = jnp.where(dec, i - 1, jnp.where(inc, i + 1, i))
    return i


@functools.partial(
    pl.kernel,
    out_type=jax.ShapeDtypeStruct((2 * BATCH, EMBED_DIM), jnp.float32),
    mesh=plsc.VectorSubcoreMesh(
        core_axis_name="c", subcore_axis_name="s", num_cores=NC, num_subcores=NS
    ),
    compiler_params=pltpu.CompilerParams(
        needs_layout_passes=False, use_tc_tiling_on_sc=False
    ),
    scratch_types=[
        pltpu.VMEM((CHUNK,), jnp.float32),
        pltpu.VMEM((CHUNK,), jnp.float32),
        pltpu.VMEM((2, _PAD), jnp.float32),
        pltpu.VMEM((NGATHER, GATHER), jnp.int32),
        pltpu.VMEM((2 * CHUNK, EMBED_DIM), jnp.float32),
        pltpu.VMEM_SHARED((2 * BINS, EMBED_DIM), jnp.float32),
        pltpu.SemaphoreType.DMA,
        pltpu.SemaphoreType.DMA,
        pltpu.SemaphoreType.DMA,
        pltpu.SemaphoreType.DMA,
        pltpu.SemaphoreType.DMA,
        pltpu.SemaphoreType.DMA,
    ],
)
def _embed_sc(lat, lon, lat_table, lon_table, bks, out,
              lat_v, lon_v, bk_v, idx_v, rows_v, table_v,
              in_sem, g0, g1, g2, g3, out_sem):
    sid = lax.axis_index("s")
    wid = sid * NC + lax.axis_index("c")
    base = wid * CHUNK
    gsems = [g0, g1, g2, g3]

    tbl_cps = [
        pltpu.make_async_copy(lat_table, table_v.at[pl.ds(0, BINS)], in_sem),
        pltpu.make_async_copy(lon_table, table_v.at[pl.ds(BINS, BINS)], in_sem),
    ]

    @pl.when(sid == 0)
    def _stage_table():
        for cp in tbl_cps:
            cp.start()

    in_cps = [
        pltpu.make_async_copy(lat.at[pl.ds(base, CHUNK)], lat_v, in_sem),
        pltpu.make_async_copy(lon.at[pl.ds(base, CHUNK)], lon_v, in_sem),
        pltpu.make_async_copy(bks, bk_v, in_sem),
    ]
    for cp in in_cps:
        cp.start()
    for cp in in_cps:
        cp.wait()

    lane2 = 2 * lax.iota(jnp.int32, L)
    for j in range(NGATHER):
        for k in range(GROUPS_PER_GATHER):
            g = GROUPS_PER_GATHER * j + k
            x_lat = lat_v[pl.ds(g * L, L)]
            x_lon = lon_v[pl.ds(g * L, L)]
            i_lat = _digitize(x_lat, bk_v.at[0], LAT_MIN, LAT_MAX)
            i_lon = _digitize(x_lon, bk_v.at[1], LON_MIN, LON_MAX) + BINS
            col = 2 * L * k + lane2
            plsc.store_scatter(idx_v.at[j], [col], i_lat)
            plsc.store_scatter(idx_v.at[j], [col + 1], i_lon)

    @pl.when(sid == 0)
    def _wait_table():
        for cp in tbl_cps:
            cp.wait()

    plsc.subcore_barrier()

    gather_cps = []
    for j in range(NGATHER):
        cp = pltpu.make_async_copy(
            table_v.at[idx_v.at[j]],
            rows_v.at[pl.ds(j * GATHER, GATHER)],
            gsems[j // 2],
        )
        cp.start()
        gather_cps.append(cp)

    out_cps = []
    for p in range(NPAIR):
        gather_cps[2 * p].wait()
        gather_cps[2 * p + 1].wait()
        cp = pltpu.make_async_copy(
            rows_v.at[pl.ds(p * 2 * GATHER, 2 * GATHER)],
            out.at[pl.ds(2 * base + p * 2 * GATHER, 2 * GATHER)],
            out_sem,
        )
        cp.start()
        out_cps.append(cp)
    for cp in out_cps:
        cp.wait()


def kernel(lat, lon, lat_table, lon_table):
    out = _embed_sc(lat, lon, lat_table, lon_table, jnp.asarray(BKS))
    return out.reshape(BATCH, 2 * EMBED_DIM)


# speedup vs baseline: 1.0461x; 1.0461x over previous
import functools

import jax
import jax.numpy as jnp
import numpy as np
from jax import lax
from jax.experimental import pallas as pl
from jax.experimental.pallas import tpu as pltpu
from jax.experimental.pallas import tpu_sc as plsc

LAT_MIN, LAT_MAX = -90.0, 90.0
LON_MIN, LON_MAX = -180.0, 180.0
BINS = 100
EMBED_DIM = 64
BATCH = 16384

NC, NS, L = 2, 16, 16
NW = NC * NS
CHUNK = BATCH // NW
GATHER = 128
NGATHER = 2 * CHUNK // GATHER
GROUPS_PER_GATHER = GATHER // (2 * L)
NPAIR = NGATHER // 2

_PAD = 112


def _buckets(lo, hi):
    b = np.linspace(lo, hi, BINS - 1).astype(np.float32)
    return np.pad(b, (0, _PAD - (BINS - 1)), constant_values=b[-1])


BKS = np.stack([_buckets(LAT_MIN, LAT_MAX), _buckets(LON_MIN, LON_MAX)])


def _digitize(x, bk_ref, lo, hi):
    scale = float(BINS - 2) / (hi - lo)
    t = (x - lo) * scale
    t = jnp.minimum(jnp.maximum(t, -1.0), float(BINS + 1))
    i = t.astype(jnp.int32) + 1
    i = jnp.clip(i, 0, BINS - 1)
    for _ in range(1):
        b_lo = plsc.load_gather(bk_ref, [jnp.clip(i - 1, 0, BINS - 2)])
        b_hi = plsc.load_gather(bk_ref, [jnp.clip(i, 0, BINS - 2)])
        dec = (i > 0) & (b_lo > x)
        inc = (i < BINS - 1) & (b_hi <= x)
        i = jnp.where(dec, i - 1, jnp.where(inc, i + 1, i))
    return i


@functools.partial(
    pl.kernel,
    out_type=jax.ShapeDtypeStruct((2 * BATCH, EMBED_DIM), jnp.float32),
    mesh=plsc.VectorSubcoreMesh(
        core_axis_name="c", subcore_axis_name="s", num_cores=NC, num_subcores=NS
    ),
    compiler_params=pltpu.CompilerParams(
        needs_layout_passes=False, use_tc_tiling_on_sc=False
    ),
    scratch_types=[
        pltpu.VMEM((CHUNK,), jnp.float32),
        pltpu.VMEM((CHUNK,), jnp.float32),
        pltpu.VMEM((2, _PAD), jnp.float32),
        pltpu.VMEM((NGATHER, GATHER), jnp.int32),
        pltpu.VMEM((2 * CHUNK, EMBED_DIM), jnp.float32),
        pltpu.VMEM_SHARED((2 * BINS, EMBED_DIM), jnp.float32),
        pltpu.SemaphoreType.DMA,
        pltpu.SemaphoreType.DMA,
        pltpu.SemaphoreType.DMA,
        pltpu.SemaphoreType.DMA,
        pltpu.SemaphoreType.DMA,
        pltpu.SemaphoreType.DMA,
        pltpu.SemaphoreType.DMA,
    ],
)
def _embed_sc(lat, lon, lat_table, lon_table, bks, out,
              lat_v, lon_v, bk_v, idx_v, rows_v, table_v,
              in_sem, tbl_sem, g0, g1, g2, g3, out_sem):
    sid = lax.axis_index("s")
    wid = sid * NC + lax.axis_index("c")
    base = wid * CHUNK
    gsems = [g0, g1, g2, g3]

    tbl_cps = [
        pltpu.make_async_copy(lat_table, table_v.at[pl.ds(0, BINS)], tbl_sem),
        pltpu.make_async_copy(lon_table, table_v.at[pl.ds(BINS, BINS)], tbl_sem),
    ]

    @pl.when(sid == 0)
    def _stage_table():
        for cp in tbl_cps:
            cp.start()

    in_cps = [
        pltpu.make_async_copy(lat.at[pl.ds(base, CHUNK)], lat_v, in_sem),
        pltpu.make_async_copy(lon.at[pl.ds(base, CHUNK)], lon_v, in_sem),
        pltpu.make_async_copy(bks, bk_v, in_sem),
    ]
    for cp in in_cps:
        cp.start()
    for cp in in_cps:
        cp.wait()

    lane2 = 2 * lax.iota(jnp.int32, L)

    def _digitize_chunk(j):
        for k in range(GROUPS_PER_GATHER):
            g = GROUPS_PER_GATHER * j + k
            x_lat = lat_v[pl.ds(g * L, L)]
            x_lon = lon_v[pl.ds(g * L, L)]
            i_lat = _digitize(x_lat, bk_v.at[0], LAT_MIN, LAT_MAX)
            i_lon = _digitize(x_lon, bk_v.at[1], LON_MIN, LON_MAX) + BINS
            col = 2 * L * k + lane2
            plsc.store_scatter(idx_v.at[j], [col], i_lat)
            plsc.store_scatter(idx_v.at[j], [col + 1], i_lon)

    _digitize_chunk(0)

    @pl.when(sid == 0)
    def _wait_table():
        for cp in tbl_cps:
            cp.wait()

    plsc.subcore_barrier()

    gather_cps = []
    for j in range(NGATHER):
        cp = pltpu.make_async_copy(
            table_v.at[idx_v.at[j]],
            rows_v.at[pl.ds(j * GATHER, GATHER)],
            gsems[j // 2],
        )
        cp.start()
        gather_cps.append(cp)
        if j + 1 < NGATHER:
            _digitize_chunk(j + 1)

    out_cps = []
    for p in range(NPAIR):
        gather_cps[2 * p].wait()
        gather_cps[2 * p + 1].wait()
        cp = pltpu.make_async_copy(
            rows_v.at[pl.ds(p * 2 * GATHER, 2 * GATHER)],
            out.at[pl.ds(2 * base + p * 2 * GATHER, 2 * GATHER)],
            out_sem,
        )
        cp.start()
        out_cps.append(cp)
    for cp in out_cps:
        cp.wait()


def kernel(lat, lon, lat_table, lon_table):
    out = _embed_sc(lat, lon, lat_table, lon_table, jnp.asarray(BKS))
    return out.reshape(BATCH, 2 * EMBED_DIM)
